# Initial kernel scaffold; baseline (speedup 1.0000x reference)
import jax
import jax.numpy as jnp
from jax.experimental import pallas as pl


def kernel(x, table):
    raise NotImplementedError("write your pallas kernel here")


# speedup vs baseline: 6.3364x; 6.3364x over previous
import functools

import jax
import jax.numpy as jnp
from jax import lax
from jax.experimental import pallas as pl
from jax.experimental.pallas import tpu as pltpu
from jax.experimental.pallas import tpu_sc as plsc

_D = 128
_CH = 128
_NC = 2
_NS = 16


@functools.lru_cache(maxsize=None)
def _build(n_tokens: int, vocab: int):
    nw = _NC * _NS
    per_w = n_tokens // nw
    nch = per_w // _CH
    assert per_w * nw == n_tokens and nch * _CH == per_w

    mesh = plsc.VectorSubcoreMesh(core_axis_name="c", subcore_axis_name="s")

    @functools.partial(
        pl.kernel,
        mesh=mesh,
        out_type=jax.ShapeDtypeStruct((n_tokens, _D), jnp.float32),
        scratch_types=[
            pltpu.VMEM((nch, _CH), jnp.int32),
            pltpu.VMEM((_CH, _D), jnp.float32),
            pltpu.SemaphoreType.DMA,
        ],
    )
    def emb(idx_hbm, tab_hbm, out_hbm, idx_v, rows, gsem):
        wid = lax.axis_index("s") * _NC + lax.axis_index("c")
        base = wid * per_w
        pltpu.sync_copy(idx_hbm.at[wid], idx_v)

        def body(j, carry):
            pltpu.async_copy(tab_hbm.at[idx_v.at[j]], rows, gsem).wait()
            pltpu.sync_copy(rows, out_hbm.at[pl.ds(base + j * _CH, _CH)])
            return carry

        lax.fori_loop(0, nch, body, 0)

    return emb


def kernel(x, table):
    b, s = x.shape
    n = b * s
    idx = x.reshape(_NC * _NS, n // (_NC * _NS) // _CH, _CH).astype(jnp.int32)
    out = _build(n, table.shape[0])(idx, table)
    return out.reshape(b, s, _D)


# speedup vs baseline: 9.2803x; 1.4646x over previous
import functools

import jax
import jax.numpy as jnp
from jax import lax
from jax.experimental import pallas as pl
from jax.experimental.pallas import tpu as pltpu
from jax.experimental.pallas import tpu_sc as plsc

_D = 128
_CH = 128
_NC = 2
_NS = 16
_NBUF = 5


@functools.lru_cache(maxsize=None)
def _build(n_tokens: int):
    nw = _NC * _NS
    per_w = n_tokens // nw
    nch = per_w // _CH
    assert per_w * nw == n_tokens and nch * _CH == per_w
    assert nch % _NBUF == 0 and nch >= 3 * _NBUF

    mesh = plsc.VectorSubcoreMesh(core_axis_name="c", subcore_axis_name="s")

    @functools.partial(
        pl.kernel,
        mesh=mesh,
        out_type=jax.ShapeDtypeStruct((n_tokens, _D), jnp.float32),
        scratch_types=[
            pltpu.VMEM((nch, _CH), jnp.int32),
            [pltpu.VMEM((_CH, _D), jnp.float32) for _ in range(_NBUF)],
            [pltpu.SemaphoreType.DMA for _ in range(_NBUF)],
            [pltpu.SemaphoreType.DMA for _ in range(_NBUF)],
        ],
    )
    def emb(idx_hbm, tab_hbm, out_hbm, idx_v, rows, gsem, osem):
        wid = lax.axis_index("s") * _NC + lax.axis_index("c")
        base = wid * per_w
        pltpu.sync_copy(idx_hbm.at[wid], idx_v)

        def start_g(j, b):
            pltpu.async_copy(tab_hbm.at[idx_v.at[j]], rows[b], gsem[b])

        def wait_g(b):
            pltpu.make_async_copy(tab_hbm.at[idx_v.at[0]], rows[b], gsem[b]).wait()

        def start_w(j, b):
            pltpu.async_copy(rows[b], out_hbm.at[pl.ds(base + j * _CH, _CH)],
                             osem[b])

        def wait_w(b):
            pltpu.make_async_copy(rows[b], out_hbm.at[pl.ds(base, _CH)],
                                  osem[b]).wait()

        def step(j, b, first, last):
            bn = (b - 1) % _NBUF
            wait_g(b)
            start_w(j, b)
            if not first:
                wait_w(bn)
            if not last:
                start_g(j + _NBUF - 1, bn)

        for b in range(_NBUF - 1):
            start_g(b, b)
        for b in range(_NBUF):
            step(b, b, first=(b == 0), last=False)

        @pl.loop(_NBUF, nch - _NBUF, step=_NBUF)
        def _(g):
            for b in range(_NBUF):
                step(g + b, b, first=False, last=False)

        for b in range(_NBUF):
            step(nch - _NBUF + b, b, first=False, last=(b != 0))
        wait_w(_NBUF - 1)

    return emb


def kernel(x, table):
    b, s = x.shape
    n = b * s
    idx = x.reshape(_NC * _NS, n // (_NC * _NS) // _CH, _CH).astype(jnp.int32)
    out = _build(n)(idx, table)
    return out.reshape(b, s, _D)
